# baseline (speedup 1.0000x reference)
import functools

import jax
import jax.numpy as jnp
from jax import lax
from jax.experimental import pallas as pl

_INTERPRET = False

GS = 1024
E = 8
K = 2
CAP = 256


def _router_kernel(x_ref, wr_ref, topi_ref, pos_ref, keep_ref, cgate_ref, aux_ref,
                   *, G):
    x = x_ref[...]
    wr = wr_ref[...]
    N = G * GS
    logits = jnp.dot(x, wr, preferred_element_type=jnp.float32)
    m = jnp.max(logits, axis=-1, keepdims=True)
    ex = jnp.exp(logits - m)
    gates = ex / jnp.sum(ex, axis=-1, keepdims=True)

    idx8 = lax.broadcasted_iota(jnp.int32, (N, E), 1)
    top1v = jnp.max(gates, axis=-1, keepdims=True)
    top1i = jnp.min(jnp.where(gates == top1v, idx8, E), axis=-1, keepdims=True)
    oh1 = idx8 == top1i
    masked = jnp.where(oh1, -1.0, gates)
    top2v = jnp.max(masked, axis=-1, keepdims=True)
    top2i = jnp.min(jnp.where(masked == top2v, idx8, E), axis=-1, keepdims=True)
    oh2 = idx8 == top2i

    M = jnp.concatenate([oh1.astype(jnp.float32).reshape(G, GS, E),
                         oh2.astype(jnp.float32).reshape(G, GS, E)], axis=1)
    C = M
    sh = 1
    while sh < K * GS:
        C = C + jnp.concatenate(
            [jnp.zeros((G, sh, E), jnp.float32), C[:, :-sh, :]], axis=1)
        sh *= 2
    P = C - 1.0
    pos_entry = jnp.sum(M * P, axis=-1)
    keep = (pos_entry < CAP).astype(jnp.float32)
    pos_i = jnp.minimum(pos_entry, CAP - 1).astype(jnp.int32)

    topi_flat = jnp.concatenate([top1i.reshape(G, GS), top2i.reshape(G, GS)],
                                axis=1)
    gsel = jnp.concatenate([top1v.reshape(G, GS), top2v.reshape(G, GS)], axis=1)

    topi_ref[...] = topi_flat.reshape(G, 1, K * GS)
    pos_ref[...] = pos_i.reshape(G, 1, K * GS)
    keep_ref[...] = keep.reshape(G, 1, K * GS)
    cgate_ref[...] = (gsel * keep).reshape(G, 1, K * GS)

    imp = jnp.sum(gates.reshape(G, GS, E), axis=1)
    mu = jnp.mean(imp, axis=-1, keepdims=True)
    var = jnp.mean((imp - mu) ** 2, axis=-1, keepdims=True)
    aux_ref[...] = jnp.mean(var / (mu + 1e-10) ** 2).reshape(1, 1)


def _mlp_kernel(x_ref, w1_ref, b1_ref, w2_ref, b2_ref,
                topi_ref, pos_ref, keep_ref, y_ref):
    e = pl.program_id(0)
    t = topi_ref[0]
    p = pos_ref[0]
    kp = keep_ref[0]
    t0, t1 = t[:, :GS], t[:, GS:]
    p0, p1 = p[:, :GS], p[:, GS:]
    k0, k1 = kp[:, :GS], kp[:, GS:]

    ci = lax.broadcasted_iota(jnp.int32, (CAP, GS), 0)
    oh0 = ((p0 == ci) & (t0 == e) & (k0 > 0)).astype(jnp.bfloat16)
    oh1 = ((p1 == ci) & (t1 == e) & (k1 > 0)).astype(jnp.bfloat16)
    dispT = oh0 + oh1

    xe = jnp.dot(dispT, x_ref[0], preferred_element_type=jnp.float32)
    h = jnp.dot(xe.astype(jnp.bfloat16), w1_ref[0],
                preferred_element_type=jnp.float32) + b1_ref[0]
    h = jax.nn.gelu(h)
    y = jnp.dot(h.astype(jnp.bfloat16), w2_ref[0],
                preferred_element_type=jnp.float32) + b2_ref[0]
    y_ref[0, 0] = y.astype(jnp.bfloat16)


def _combine_kernel(y_ref, topi_ref, pos_ref, cgate_ref, out_ref):
    t = topi_ref[0]
    p = pos_ref[0]
    cg = cgate_ref[0]
    t0, t1 = t[:, :GS], t[:, GS:]
    p0, p1 = p[:, :GS], p[:, GS:]
    c0, c1 = cg[:, :GS].astype(jnp.bfloat16), cg[:, GS:].astype(jnp.bfloat16)

    ci = lax.broadcasted_iota(jnp.int32, (CAP, GS), 0)
    acc = jnp.zeros((GS, y_ref.shape[-1]), jnp.float32)
    for e in range(E):
        m0 = ((p0 == ci) & (t0 == e)).astype(jnp.bfloat16)
        m1 = ((p1 == ci) & (t1 == e)).astype(jnp.bfloat16)
        combT = m0 * c0 + m1 * c1
        acc = acc + lax.dot_general(combT, y_ref[e, 0],
                                    (((0,), (0,)), ((), ())),
                                    preferred_element_type=jnp.float32)
    out_ref[0] = acc


def kernel(inputs, w_router, w1, b1, w2, b2):
    b, s, d = inputs.shape
    G = (b * s) // GS
    MLP = w1.shape[2]
    x2 = inputs.reshape(G * GS, d)

    topi, pos, keep, cgate, aux = pl.pallas_call(
        functools.partial(_router_kernel, G=G),
        out_shape=[
            jax.ShapeDtypeStruct((G, 1, K * GS), jnp.int32),
            jax.ShapeDtypeStruct((G, 1, K * GS), jnp.int32),
            jax.ShapeDtypeStruct((G, 1, K * GS), jnp.float32),
            jax.ShapeDtypeStruct((G, 1, K * GS), jnp.float32),
            jax.ShapeDtypeStruct((1, 1), jnp.float32),
        ],
        interpret=_INTERPRET,
    )(x2, w_router)

    x3 = inputs.reshape(G, GS, d).astype(jnp.bfloat16)
    w1b = w1.astype(jnp.bfloat16)
    w2b = w2.astype(jnp.bfloat16)
    b1r = b1.reshape(E, 1, MLP)
    b2r = b2.reshape(E, 1, d)

    y = pl.pallas_call(
        _mlp_kernel,
        grid=(E, G),
        in_specs=[
            pl.BlockSpec((1, GS, d), lambda e, g: (g, 0, 0)),
            pl.BlockSpec((1, d, MLP), lambda e, g: (e, 0, 0)),
            pl.BlockSpec((1, 1, MLP), lambda e, g: (e, 0, 0)),
            pl.BlockSpec((1, MLP, d), lambda e, g: (e, 0, 0)),
            pl.BlockSpec((1, 1, d), lambda e, g: (e, 0, 0)),
            pl.BlockSpec((1, 1, K * GS), lambda e, g: (g, 0, 0)),
            pl.BlockSpec((1, 1, K * GS), lambda e, g: (g, 0, 0)),
            pl.BlockSpec((1, 1, K * GS), lambda e, g: (g, 0, 0)),
        ],
        out_specs=pl.BlockSpec((1, 1, CAP, d), lambda e, g: (e, g, 0, 0)),
        out_shape=jax.ShapeDtypeStruct((E, G, CAP, d), jnp.bfloat16),
        interpret=_INTERPRET,
    )(x3, w1b, b1r, w2b, b2r, topi, pos, keep)

    out3 = pl.pallas_call(
        _combine_kernel,
        grid=(G,),
        in_specs=[
            pl.BlockSpec((E, 1, CAP, d), lambda g: (0, g, 0, 0)),
            pl.BlockSpec((1, 1, K * GS), lambda g: (g, 0, 0)),
            pl.BlockSpec((1, 1, K * GS), lambda g: (g, 0, 0)),
            pl.BlockSpec((1, 1, K * GS), lambda g: (g, 0, 0)),
        ],
        out_specs=pl.BlockSpec((1, GS, d), lambda g: (g, 0, 0)),
        out_shape=jax.ShapeDtypeStruct((G, GS, d), jnp.float32),
        interpret=_INTERPRET,
    )(y, topi, pos, cgate)

    out = out3.reshape(b, s, d)
    aux_s = aux[0, 0]
    return out, {"auxiliary_loss": aux_s, "importance_loss": aux_s}


# speedup vs baseline: 1.2087x; 1.2087x over previous
import functools

import jax
import jax.numpy as jnp
from jax import lax
from jax.experimental import pallas as pl

_INTERPRET = False

GS = 1024
E = 8
K = 2
CAP = 256


def _router_kernel(x_ref, wr_ref, topi_ref, pos_ref, keep_ref, cgate_ref, aux_ref,
                   *, G):
    x = x_ref[...]
    wr = wr_ref[...]
    N = G * GS
    logits = jnp.dot(x, wr, preferred_element_type=jnp.float32)
    m = jnp.max(logits, axis=-1, keepdims=True)
    ex = jnp.exp(logits - m)
    gates = ex / jnp.sum(ex, axis=-1, keepdims=True)

    idx8 = lax.broadcasted_iota(jnp.int32, (N, E), 1)
    top1v = jnp.max(gates, axis=-1, keepdims=True)
    top1i = jnp.min(jnp.where(gates == top1v, idx8, E), axis=-1, keepdims=True)
    oh1 = idx8 == top1i
    masked = jnp.where(oh1, -1.0, gates)
    top2v = jnp.max(masked, axis=-1, keepdims=True)
    top2i = jnp.min(jnp.where(masked == top2v, idx8, E), axis=-1, keepdims=True)
    oh2 = idx8 == top2i

    M = jnp.concatenate([oh1.astype(jnp.float32).reshape(G, GS, E),
                         oh2.astype(jnp.float32).reshape(G, GS, E)], axis=1)
    C = M
    sh = 1
    while sh < K * GS:
        C = C + jnp.concatenate(
            [jnp.zeros((G, sh, E), jnp.float32), C[:, :-sh, :]], axis=1)
        sh *= 2
    P = C - 1.0
    pos_entry = jnp.sum(M * P, axis=-1)
    keep = (pos_entry < CAP).astype(jnp.float32)
    pos_i = jnp.minimum(pos_entry, CAP - 1).astype(jnp.int32)

    topi_flat = jnp.concatenate([top1i.reshape(G, GS), top2i.reshape(G, GS)],
                                axis=1)
    gsel = jnp.concatenate([top1v.reshape(G, GS), top2v.reshape(G, GS)], axis=1)

    topi_ref[...] = topi_flat.reshape(G, 1, K * GS)
    pos_ref[...] = pos_i.reshape(G, 1, K * GS)
    keep_ref[...] = keep.reshape(G, 1, K * GS)
    cgate_ref[...] = (gsel * keep).reshape(G, 1, K * GS)

    imp = jnp.sum(gates.reshape(G, GS, E), axis=1)
    mu = jnp.mean(imp, axis=-1, keepdims=True)
    var = jnp.mean((imp - mu) ** 2, axis=-1, keepdims=True)
    aux_ref[...] = jnp.mean(var / (mu + 1e-10) ** 2).reshape(1, 1)


def _mlp_kernel(x_ref, w1_ref, b1_ref, w2_ref, b2_ref,
                topi_ref, pos_ref, keep_ref, y_ref):
    e = pl.program_id(0)
    t = topi_ref[0]
    p = pos_ref[0]
    kp = keep_ref[0]
    t0, t1 = t[:, :GS], t[:, GS:]
    p0, p1 = p[:, :GS], p[:, GS:]
    k0, k1 = kp[:, :GS], kp[:, GS:]

    ci = lax.broadcasted_iota(jnp.int32, (CAP, GS), 0)
    oh0 = ((p0 == ci) & (t0 == e) & (k0 > 0)).astype(jnp.bfloat16)
    oh1 = ((p1 == ci) & (t1 == e) & (k1 > 0)).astype(jnp.bfloat16)
    dispT = oh0 + oh1

    xe = jnp.dot(dispT, x_ref[0], preferred_element_type=jnp.float32)
    h = jnp.dot(xe, w1_ref[0], preferred_element_type=jnp.float32) + b1_ref[0]
    h = jax.nn.gelu(h)
    y = jnp.dot(h, w2_ref[0], preferred_element_type=jnp.float32) + b2_ref[0]
    y_ref[0, 0] = y.astype(jnp.bfloat16)


def _combine_kernel(y_ref, topi_ref, pos_ref, cgate_ref, out_ref):
    t = topi_ref[0]
    p = pos_ref[0]
    cg = cgate_ref[0]
    t0, t1 = t[:, :GS], t[:, GS:]
    p0, p1 = p[:, :GS], p[:, GS:]
    c0, c1 = cg[:, :GS].astype(jnp.bfloat16), cg[:, GS:].astype(jnp.bfloat16)

    ci = lax.broadcasted_iota(jnp.int32, (CAP, GS), 0)
    acc = jnp.zeros((GS, y_ref.shape[-1]), jnp.float32)
    for e in range(E):
        m0 = ((p0 == ci) & (t0 == e)).astype(jnp.bfloat16)
        m1 = ((p1 == ci) & (t1 == e)).astype(jnp.bfloat16)
        combT = m0 * c0 + m1 * c1
        acc = acc + lax.dot_general(combT, y_ref[e, 0],
                                    (((0,), (0,)), ((), ())),
                                    preferred_element_type=jnp.float32)
    out_ref[0] = acc


def kernel(inputs, w_router, w1, b1, w2, b2):
    b, s, d = inputs.shape
    G = (b * s) // GS
    MLP = w1.shape[2]
    x2 = inputs.reshape(G * GS, d)

    topi, pos, keep, cgate, aux = pl.pallas_call(
        functools.partial(_router_kernel, G=G),
        out_shape=[
            jax.ShapeDtypeStruct((G, 1, K * GS), jnp.int32),
            jax.ShapeDtypeStruct((G, 1, K * GS), jnp.int32),
            jax.ShapeDtypeStruct((G, 1, K * GS), jnp.float32),
            jax.ShapeDtypeStruct((G, 1, K * GS), jnp.float32),
            jax.ShapeDtypeStruct((1, 1), jnp.float32),
        ],
        interpret=_INTERPRET,
    )(x2, w_router)

    x3 = inputs.reshape(G, GS, d).astype(jnp.bfloat16)
    b1r = b1.reshape(E, 1, MLP)
    b2r = b2.reshape(E, 1, d)

    y = pl.pallas_call(
        _mlp_kernel,
        grid=(E, G),
        in_specs=[
            pl.BlockSpec((1, GS, d), lambda e, g: (g, 0, 0)),
            pl.BlockSpec((1, d, MLP), lambda e, g: (e, 0, 0)),
            pl.BlockSpec((1, 1, MLP), lambda e, g: (e, 0, 0)),
            pl.BlockSpec((1, MLP, d), lambda e, g: (e, 0, 0)),
            pl.BlockSpec((1, 1, d), lambda e, g: (e, 0, 0)),
            pl.BlockSpec((1, 1, K * GS), lambda e, g: (g, 0, 0)),
            pl.BlockSpec((1, 1, K * GS), lambda e, g: (g, 0, 0)),
            pl.BlockSpec((1, 1, K * GS), lambda e, g: (g, 0, 0)),
        ],
        out_specs=pl.BlockSpec((1, 1, CAP, d), lambda e, g: (e, g, 0, 0)),
        out_shape=jax.ShapeDtypeStruct((E, G, CAP, d), jnp.bfloat16),
        interpret=_INTERPRET,
    )(x3, w1, b1r, w2, b2r, topi, pos, keep)

    out3 = pl.pallas_call(
        _combine_kernel,
        grid=(G,),
        in_specs=[
            pl.BlockSpec((E, 1, CAP, d), lambda g: (0, g, 0, 0)),
            pl.BlockSpec((1, 1, K * GS), lambda g: (g, 0, 0)),
            pl.BlockSpec((1, 1, K * GS), lambda g: (g, 0, 0)),
            pl.BlockSpec((1, 1, K * GS), lambda g: (g, 0, 0)),
        ],
        out_specs=pl.BlockSpec((1, GS, d), lambda g: (g, 0, 0)),
        out_shape=jax.ShapeDtypeStruct((G, GS, d), jnp.float32),
        interpret=_INTERPRET,
    )(y, topi, pos, cgate)

    out = out3.reshape(b, s, d)
    aux_s = aux[0, 0]
    return out, {"auxiliary_loss": aux_s, "importance_loss": aux_s}
